# Initial kernel scaffold; baseline (speedup 1.0000x reference)
import jax
import jax.numpy as jnp
from jax.experimental import pallas as pl


def kernel(node_features, edge_features, senders, receivers, params):
    raise NotImplementedError("write your pallas kernel here")


# speedup vs baseline: 2.4658x; 2.4658x over previous
import functools

import jax
import jax.numpy as jnp
from jax import lax
from jax.experimental import pallas as pl
from jax.experimental.pallas import tpu as pltpu
from jax.experimental.pallas import tpu_sc as plsc

N_NODES = 10000
N_EDGES = 160000
D_NODE = 128
D_EDGE = 16
LATENT = 128
OUT_SIZE = 3

NP = 10240
EP = 163840
IDX_ROWS = 1280
NCORES = 2
NSUB = 16
NW = NCORES * NSUB
ROWS_PER_W = IDX_ROWS // NW
ROWS_PER_CORE = IDX_ROWS // NCORES

_mesh = plsc.VectorSubcoreMesh(
    core_axis_name="c", subcore_axis_name="s", num_cores=NCORES, num_subcores=NSUB
)


@functools.partial(
    pl.kernel,
    out_type=[
        jax.ShapeDtypeStruct((EP, LATENT), jnp.float32),
        jax.ShapeDtypeStruct((EP, LATENT), jnp.float32),
    ],
    mesh=_mesh,
    scratch_types=[
        pltpu.VMEM((ROWS_PER_W, 128), jnp.int32),
        pltpu.VMEM((ROWS_PER_W, 128), jnp.int32),
        pltpu.VMEM((128, LATENT), jnp.float32),
        pltpu.VMEM((128, LATENT), jnp.float32),
        pltpu.SemaphoreType.DMA,
        pltpu.SemaphoreType.DMA,
    ],
)
def _gather(x_hbm, sidx_hbm, ridx_hbm, sf_hbm, rf_hbm,
            sidx_v, ridx_v, buf_s, buf_r, sem_s, sem_r):
    wid = lax.axis_index("s") * NCORES + lax.axis_index("c")
    base = wid * ROWS_PER_W
    pltpu.sync_copy(sidx_hbm.at[pl.ds(base, ROWS_PER_W)], sidx_v)
    pltpu.sync_copy(ridx_hbm.at[pl.ds(base, ROWS_PER_W)], ridx_v)

    @pl.loop(0, ROWS_PER_W)
    def _(j):
        cs = pltpu.async_copy(x_hbm.at[sidx_v.at[j]], buf_s, sem_s)
        cr = pltpu.async_copy(x_hbm.at[ridx_v.at[j]], buf_r, sem_r)
        cs.wait()
        ws = pltpu.async_copy(buf_s, sf_hbm.at[pl.ds((base + j) * 128, 128)], sem_s)
        cr.wait()
        wr = pltpu.async_copy(buf_r, rf_hbm.at[pl.ds((base + j) * 128, 128)], sem_r)
        ws.wait()
        wr.wait()


@functools.partial(
    pl.kernel,
    out_type=jax.ShapeDtypeStruct((NCORES, NP, LATENT), jnp.float32),
    mesh=_mesh,
    scratch_types=[
        pltpu.VMEM((ROWS_PER_W, 128), jnp.int32),
        pltpu.VMEM((128, LATENT), jnp.float32),
        pltpu.VMEM_SHARED((NP, LATENT), jnp.float32),
        pltpu.SemaphoreType.DMA,
    ],
)
def _scatter_add(ne_hbm, ridx_hbm, zeros_hbm, out_hbm, idx_v, buf, acc, sem):
    cid = lax.axis_index("c")
    sid = lax.axis_index("s")
    rows_per_sub = NP // NSUB
    pltpu.sync_copy(zeros_hbm.at[pl.ds(sid * rows_per_sub, rows_per_sub)],
                    acc.at[pl.ds(sid * rows_per_sub, rows_per_sub)])
    plsc.subcore_barrier()

    base = cid * ROWS_PER_CORE + sid * ROWS_PER_W
    pltpu.sync_copy(ridx_hbm.at[pl.ds(base, ROWS_PER_W)], idx_v)

    @pl.loop(0, ROWS_PER_W)
    def _(j):
        pltpu.sync_copy(ne_hbm.at[pl.ds((base + j) * 128, 128)], buf)
        pltpu.sync_copy(buf, acc.at[idx_v.at[j]], add=True)

    plsc.subcore_barrier()
    pltpu.sync_copy(acc.at[pl.ds(sid * rows_per_sub, rows_per_sub)],
                    out_hbm.at[cid].at[pl.ds(sid * rows_per_sub, rows_per_sub)])


def _ln(o, g, b):
    mu = jnp.mean(o, axis=-1, keepdims=True)
    var = jnp.mean((o - mu) * (o - mu), axis=-1, keepdims=True)
    return (o - mu) * lax.rsqrt(var + 1e-5) * g + b


def _dot(a, w):
    return jnp.dot(a, w, preferred_element_type=jnp.float32)


def _enc_kernel(x_ref, w1, b1, w2, b2, w3, b3, g, bt, o_ref):
    h = jnp.maximum(_dot(x_ref[...], w1[...]) + b1[...], 0.0)
    h = jnp.maximum(_dot(h, w2[...]) + b2[...], 0.0)
    o = _dot(h, w3[...]) + b3[...]
    o_ref[...] = _ln(o, g[...], bt[...])


def _edge_kernel(sf, rf, e, w1a, w1b, w1c, b1, w2, b2, w3, b3, g, bt,
                 ne_ref, eo_ref):
    h = _dot(sf[...], w1a[...]) + _dot(rf[...], w1b[...]) + _dot(e[...], w1c[...])
    h = jnp.maximum(h + b1[...], 0.0)
    h = jnp.maximum(_dot(h, w2[...]) + b2[...], 0.0)
    o = _dot(h, w3[...]) + b3[...]
    ne = _ln(o, g[...], bt[...])
    ne_ref[...] = ne
    eo_ref[...] = e[...] + ne


def _node_kernel(x, a0, a1, w1a, w1b, b1, w2, b2, w3, b3, g, bt, xo_ref):
    agg = a0[...] + a1[...]
    h = _dot(x[...], w1a[...]) + _dot(agg, w1b[...])
    h = jnp.maximum(h + b1[...], 0.0)
    h = jnp.maximum(_dot(h, w2[...]) + b2[...], 0.0)
    o = _dot(h, w3[...]) + b3[...]
    xo_ref[...] = x[...] + _ln(o, g[...], bt[...])


def _dec_kernel(x_ref, w1, b1, w2, b2, w3, b3, o_ref):
    h = jnp.maximum(_dot(x_ref[...], w1[...]) + b1[...], 0.0)
    h = jnp.maximum(_dot(h, w2[...]) + b2[...], 0.0)
    o_ref[...] = _dot(h, w3[...]) + b3[...]


def _full(shape):
    return pl.BlockSpec(shape, lambda i: tuple(0 for _ in shape))


def _rows(n_rows, blk, d):
    return pl.BlockSpec((blk, d), lambda i: (i, 0))


def _row_call(kfn, n_rows, blk, n_out, extra_specs, out_d=LATENT):
    grid = n_rows // blk
    out_shape = [jax.ShapeDtypeStruct((n_rows, out_d), jnp.float32)] * n_out
    out_specs = [pl.BlockSpec((blk, out_d), lambda i: (i, 0))] * n_out
    return pl.pallas_call(
        kfn,
        grid=grid,
        in_specs=extra_specs,
        out_specs=out_specs if n_out > 1 else out_specs[0],
        out_shape=out_shape if n_out > 1 else out_shape[0],
    )


def _wspecs(ws):
    return [_full(w.shape) for w in ws]


def _prep_mlp(mlp):
    out = []
    for w, b in mlp:
        out.append(w)
        out.append(b.reshape(1, -1))
    return out


def kernel(node_features, edge_features, senders, receivers, params):
    f32 = jnp.float32
    x_in = jnp.zeros((NP, D_NODE), f32).at[:N_NODES].set(node_features)
    ef_in = jnp.zeros((EP, D_EDGE), f32).at[:N_EDGES].set(edge_features)
    sidx = jnp.zeros((EP,), jnp.int32).at[:N_EDGES].set(senders).reshape(IDX_ROWS, 128)
    ridx = jnp.full((EP,), N_NODES, jnp.int32).at[:N_EDGES].set(receivers).reshape(IDX_ROWS, 128)
    zeros_acc = jnp.zeros((NP, LATENT), f32)

    BLK_E = 2048
    BLK_N = 2048

    enc_n = params["enc_node"]
    ws = _prep_mlp(enc_n["mlp"]) + [enc_n["ln"][0].reshape(1, -1), enc_n["ln"][1].reshape(1, -1)]
    x = _row_call(_enc_kernel, NP, BLK_N, 1,
                  [_rows(NP, BLK_N, D_NODE)] + _wspecs(ws))(x_in, *ws)

    enc_e = params["enc_edge"]
    ws = _prep_mlp(enc_e["mlp"]) + [enc_e["ln"][0].reshape(1, -1), enc_e["ln"][1].reshape(1, -1)]
    e = _row_call(_enc_kernel, EP, BLK_E, 1,
                  [_rows(EP, BLK_E, D_EDGE)] + _wspecs(ws))(ef_in, *ws)

    for blk in params["blocks"]:
        sf, rf = _gather(x, sidx, ridx)

        em = blk["edge"]["mlp"]
        w1 = em[0][0]
        ews = [w1[:LATENT], w1[LATENT:2 * LATENT], w1[2 * LATENT:], em[0][1].reshape(1, -1),
               em[1][0], em[1][1].reshape(1, -1), em[2][0], em[2][1].reshape(1, -1),
               blk["edge"]["ln"][0].reshape(1, -1), blk["edge"]["ln"][1].reshape(1, -1)]
        ne, e = _row_call(
            _edge_kernel, EP, BLK_E, 2,
            [_rows(EP, BLK_E, LATENT)] * 3 + _wspecs(ews))(sf, rf, e, *ews)

        aggs = _scatter_add(ne, ridx, zeros_acc)

        nm = blk["node"]["mlp"]
        w1 = nm[0][0]
        nws = [w1[:LATENT], w1[LATENT:], nm[0][1].reshape(1, -1),
               nm[1][0], nm[1][1].reshape(1, -1), nm[2][0], nm[2][1].reshape(1, -1),
               blk["node"]["ln"][0].reshape(1, -1), blk["node"]["ln"][1].reshape(1, -1)]
        x = _row_call(
            _node_kernel, NP, BLK_N, 1,
            [_rows(NP, BLK_N, LATENT)] * 3 + _wspecs(nws))(x, aggs[0], aggs[1], *nws)

    dm = params["dec"]["mlp"]
    w3 = jnp.zeros((LATENT, 128), f32).at[:, :OUT_SIZE].set(dm[2][0])
    b3 = jnp.zeros((1, 128), f32).at[:, :OUT_SIZE].set(dm[2][1])
    dws = [dm[0][0], dm[0][1].reshape(1, -1), dm[1][0], dm[1][1].reshape(1, -1), w3, b3]
    out = _row_call(_dec_kernel, NP, BLK_N, 1,
                    [_rows(NP, BLK_N, LATENT)] + _wspecs(dws), out_d=128)(x, *dws)
    return out[:N_NODES, :OUT_SIZE]
